# baseline (speedup 1.0000x reference)
import functools

import jax
import jax.numpy as jnp
from jax import lax
from jax.experimental import pallas as pl
from jax.experimental.pallas import tpu as pltpu
from jax.experimental.pallas import tpu_sc as plsc

EMBED = 32
T = 200
B = 4096

NW = 32
BPW = B // NW
BPC = 4
NCH = BPW // BPC
CHUNK = BPC * T
W0, W1 = 128, T - 128


VOCAB = 1000000
DCOLS = 512
NBLK = (VOCAB + DCOLS - 1) // DCOLS
PVOCAB = NBLK * DCOLS


def _detile_body(t_ref, y_ref):
    s = t_ref[...].reshape(EMBED, 4, 128)
    s = s.transpose(1, 0, 2).reshape(128, 128)
    y_ref[...] = s.T


def _detile_table(tokt):
    return pl.pallas_call(
        _detile_body,
        grid=(NBLK,),
        in_specs=[pl.BlockSpec((EMBED, DCOLS), lambda i: (0, i))],
        out_specs=pl.BlockSpec((128, 128), lambda i: (i, 0)),
        out_shape=jax.ShapeDtypeStruct((PVOCAB * EMBED // 128, 128),
                                       jnp.float32),
    )(tokt)


def _perm(v):
    return ((v >> 9) << 9) | ((v & 127) << 2) | ((v >> 7) & 3)


def _repack_x_body(x_ref, y_ref):
    xp = _perm(x_ref[...])
    a = xp[:, :W0].reshape(B // 8, 8, W0)
    c = jnp.concatenate(
        [xp[:, W0:], jnp.zeros((B, W0 - W1), jnp.int32)], axis=1)
    y_ref[:, 0] = a
    y_ref[:, 1] = c.reshape(B // 8, 8, W0)


def _repack_x(x):
    return pl.pallas_call(
        _repack_x_body,
        out_shape=jax.ShapeDtypeStruct((B // 8, 2, 8, W0), jnp.int32),
    )(x)


def _sc_embed():
    mesh = plsc.VectorSubcoreMesh(core_axis_name="c", subcore_axis_name="s")

    @functools.partial(
        pl.kernel,
        mesh=mesh,
        compiler_params=pltpu.CompilerParams(use_tc_tiling_on_sc=False),
        out_type=jax.ShapeDtypeStruct((NW, NCH, CHUNK, EMBED), jnp.float32),
        scratch_types=[
            pltpu.VMEM((BPW // 8, 2, 8, W0), jnp.int32),
            pltpu.VMEM((2, CHUNK, EMBED), jnp.float32),
            pltpu.VMEM((T, EMBED), jnp.float32),
            pltpu.SemaphoreType.DMA,
            pltpu.SemaphoreType.DMA,
            pltpu.SemaphoreType.DMA,
            pltpu.SemaphoreType.DMA,
        ],
    )
    def k(table_hbm, xp_hbm, pos_hbm, out_hbm,
          idx_v, rows_v, pos_v, gsem0, gsem1, osem0, osem1):
        wid = lax.axis_index("s") * 2 + lax.axis_index("c")
        gsem = (gsem0, gsem1)
        osem = (osem0, osem1)

        def fire_gathers(b, g):
            for q in range(BPC):
                bl = BPC * g + q
                bt = bl // 8
                b8 = bl % 8
                pltpu.async_copy(
                    table_hbm.at[idx_v.at[bt, 0, b8]],
                    rows_v.at[b, pl.ds(q * T, W0)],
                    gsem[b],
                )
                pltpu.async_copy(
                    table_hbm.at[idx_v.at[bt, 1, b8, pl.ds(0, W1)]],
                    rows_v.at[b, pl.ds(q * T + W0, W1)],
                    gsem[b],
                )

        def drain_gathers(b):
            pltpu.make_async_copy(
                out_hbm.at[wid, 0], rows_v.at[b], gsem[b]).wait()

        def fire_out(b, g):
            pltpu.async_copy(rows_v.at[b], out_hbm.at[wid, g], osem[b])

        def drain_out(b):
            pltpu.make_async_copy(
                rows_v.at[b], out_hbm.at[wid, 0], osem[b]).wait()

        def pos_add(b):
            def row_body(r, c):
                p0 = pos_v[r, pl.ds(0, 16)]
                p1 = pos_v[r, pl.ds(16, 16)]
                for q in range(BPC):
                    rr = q * T + r
                    rows_v[b, rr, pl.ds(0, 16)] = (
                        rows_v[b, rr, pl.ds(0, 16)] + p0)
                    rows_v[b, rr, pl.ds(16, 16)] = (
                        rows_v[b, rr, pl.ds(16, 16)] + p1)
                return c

            lax.fori_loop(0, T, row_body, 0)

        pltpu.sync_copy(pos_hbm, pos_v)
        pltpu.sync_copy(xp_hbm.at[pl.ds(wid * (BPW // 8), BPW // 8)], idx_v)
        fire_gathers(0, 0)

        def outer(i, carry):
            for b in range(2):
                g = 2 * i + b
                pb = 1 - b

                @pl.when(g + 1 < NCH)
                def _stage():
                    @pl.when(g >= 1)
                    def _free():
                        drain_out(pb)

                    fire_gathers(pb, g + 1)

                drain_gathers(b)
                pos_add(b)
                fire_out(b, g)
            return carry

        lax.fori_loop(0, NCH // 2, outer, 0)
        drain_out(0)
        drain_out(1)

    return k


def kernel(x, token_emb, pos_emb):
    xp = _repack_x(x.astype(jnp.int32))
    table = _detile_table(token_emb.T).reshape(PVOCAB, EMBED)
    out = _sc_embed()(table, xp, pos_emb)
    return out.reshape(B, T, EMBED)


# speedup vs baseline: 1.4631x; 1.4631x over previous
import functools

import jax
import jax.numpy as jnp
from jax import lax
from jax.experimental import pallas as pl
from jax.experimental.pallas import tpu as pltpu
from jax.experimental.pallas import tpu_sc as plsc

EMBED = 32
T = 200
B = 4096

NW = 32
BPW = B // NW
BPC = 4
NCH = BPW // BPC
CHUNK = BPC * T
W0, W1 = 128, T - 128


VOCAB = 1000000
DCOLS = 2048
NBLK = (VOCAB + DCOLS - 1) // DCOLS
PVOCAB = NBLK * DCOLS


def _detile_body(t_ref, y_ref):
    ident = jnp.eye(EMBED, dtype=jnp.float32)
    for k in range(DCOLS // 512):
        for j in range(4):
            blk = t_ref[:, pl.ds(512 * k + 128 * j, 128)]
            y_ref[pl.ds(128 * k, 128), pl.ds(32 * j, 32)] = lax.dot_general(
                blk, ident, (((0,), (0,)), ((), ())),
                precision=lax.Precision.HIGHEST,
                preferred_element_type=jnp.float32)


def _detile_table(tokt):
    return pl.pallas_call(
        _detile_body,
        grid=(NBLK,),
        in_specs=[pl.BlockSpec((EMBED, DCOLS), lambda i: (0, i))],
        out_specs=pl.BlockSpec((DCOLS // 4, 128), lambda i: (i, 0)),
        out_shape=jax.ShapeDtypeStruct((PVOCAB * EMBED // 128, 128),
                                       jnp.float32),
    )(tokt)


def _perm(v):
    return ((v >> 9) << 9) | ((v & 127) << 2) | ((v >> 7) & 3)


def _repack_x_body(x_ref, y_ref):
    xp = _perm(x_ref[...])
    a = xp[:, :W0].reshape(B // 8, 8, W0)
    c = jnp.concatenate(
        [xp[:, W0:], jnp.zeros((B, W0 - W1), jnp.int32)], axis=1)
    y_ref[:, 0] = a
    y_ref[:, 1] = c.reshape(B // 8, 8, W0)


def _repack_x(x):
    return pl.pallas_call(
        _repack_x_body,
        out_shape=jax.ShapeDtypeStruct((B // 8, 2, 8, W0), jnp.int32),
    )(x)


def _sc_embed():
    mesh = plsc.VectorSubcoreMesh(core_axis_name="c", subcore_axis_name="s")

    @functools.partial(
        pl.kernel,
        mesh=mesh,
        compiler_params=pltpu.CompilerParams(use_tc_tiling_on_sc=False),
        out_type=jax.ShapeDtypeStruct((NW, NCH, CHUNK, EMBED), jnp.float32),
        scratch_types=[
            pltpu.VMEM((BPW // 8, 2, 8, W0), jnp.int32),
            pltpu.VMEM((2, CHUNK, EMBED), jnp.float32),
            pltpu.VMEM((T, EMBED), jnp.float32),
            pltpu.SemaphoreType.DMA,
            pltpu.SemaphoreType.DMA,
            pltpu.SemaphoreType.DMA,
            pltpu.SemaphoreType.DMA,
        ],
    )
    def k(table_hbm, xp_hbm, pos_hbm, out_hbm,
          idx_v, rows_v, pos_v, gsem0, gsem1, osem0, osem1):
        wid = lax.axis_index("s") * 2 + lax.axis_index("c")
        gsem = (gsem0, gsem1)
        osem = (osem0, osem1)

        def fire_gathers(b, g):
            for q in range(BPC):
                bl = BPC * g + q
                bt = bl // 8
                b8 = bl % 8
                pltpu.async_copy(
                    table_hbm.at[idx_v.at[bt, 0, b8]],
                    rows_v.at[b, pl.ds(q * T, W0)],
                    gsem[b],
                )
                pltpu.async_copy(
                    table_hbm.at[idx_v.at[bt, 1, b8, pl.ds(0, W1)]],
                    rows_v.at[b, pl.ds(q * T + W0, W1)],
                    gsem[b],
                )

        def drain_gathers(b):
            pltpu.make_async_copy(
                out_hbm.at[wid, 0], rows_v.at[b], gsem[b]).wait()

        def fire_out(b, g):
            pltpu.async_copy(rows_v.at[b], out_hbm.at[wid, g], osem[b])

        def drain_out(b):
            pltpu.make_async_copy(
                rows_v.at[b], out_hbm.at[wid, 0], osem[b]).wait()

        def pos_add(b):
            def row_body(r, c):
                p0 = pos_v[r, pl.ds(0, 16)]
                p1 = pos_v[r, pl.ds(16, 16)]
                for q in range(BPC):
                    rr = q * T + r
                    rows_v[b, rr, pl.ds(0, 16)] = (
                        rows_v[b, rr, pl.ds(0, 16)] + p0)
                    rows_v[b, rr, pl.ds(16, 16)] = (
                        rows_v[b, rr, pl.ds(16, 16)] + p1)
                return c

            lax.fori_loop(0, T, row_body, 0)

        pltpu.sync_copy(pos_hbm, pos_v)
        pltpu.sync_copy(xp_hbm.at[pl.ds(wid * (BPW // 8), BPW // 8)], idx_v)
        fire_gathers(0, 0)

        def outer(i, carry):
            for b in range(2):
                g = 2 * i + b
                pb = 1 - b

                @pl.when(g + 1 < NCH)
                def _stage():
                    @pl.when(g >= 1)
                    def _free():
                        drain_out(pb)

                    fire_gathers(pb, g + 1)

                drain_gathers(b)
                pos_add(b)
                fire_out(b, g)
            return carry

        lax.fori_loop(0, NCH // 2, outer, 0)
        drain_out(0)
        drain_out(1)

    return k


def kernel(x, token_emb, pos_emb):
    xp = _repack_x(x.astype(jnp.int32))
    table = _detile_table(token_emb.T).reshape(PVOCAB, EMBED)
    out = _sc_embed()(table, xp, pos_emb)
    return out.reshape(B, T, EMBED)


# speedup vs baseline: 1.6431x; 1.1230x over previous
import functools

import jax
import jax.numpy as jnp
from jax import lax
from jax.experimental import pallas as pl
from jax.experimental.pallas import tpu as pltpu
from jax.experimental.pallas import tpu_sc as plsc

EMBED = 32
T = 200
B = 4096

NW = 32
BPW = B // NW
BPC = 4
NCH = BPW // BPC
CHUNK = BPC * T
W0, W1 = 128, T - 128


VOCAB = 1000000
DCOLS = 8192
NBLK = (VOCAB + DCOLS - 1) // DCOLS
PVOCAB = NBLK * DCOLS


def _detile_body(t_ref, y_ref):
    ident = jnp.eye(EMBED, dtype=jnp.float32)
    for k in range(DCOLS // 512):
        for j in range(4):
            blk = t_ref[:, pl.ds(512 * k + 128 * j, 128)]
            y_ref[pl.ds(128 * k, 128), pl.ds(32 * j, 32)] = lax.dot_general(
                blk, ident, (((0,), (0,)), ((), ())),
                precision=lax.Precision.HIGHEST,
                preferred_element_type=jnp.float32)


def _detile_table(tokt):
    return pl.pallas_call(
        _detile_body,
        grid=(NBLK,),
        in_specs=[pl.BlockSpec((EMBED, DCOLS), lambda i: (0, i))],
        out_specs=pl.BlockSpec((DCOLS // 4, 128), lambda i: (i, 0)),
        out_shape=jax.ShapeDtypeStruct((PVOCAB * EMBED // 128, 128),
                                       jnp.float32),
    )(tokt)


def _perm(v):
    return ((v >> 9) << 9) | ((v & 127) << 2) | ((v >> 7) & 3)


def _repack_x_body(x_ref, y_ref):
    xp = _perm(x_ref[...])
    a = xp[:, :W0].reshape(B // 8, 8, W0)
    c = jnp.concatenate(
        [xp[:, W0:], jnp.zeros((B, W0 - W1), jnp.int32)], axis=1)
    y_ref[:, 0] = a
    y_ref[:, 1] = c.reshape(B // 8, 8, W0)


def _repack_x(x):
    return pl.pallas_call(
        _repack_x_body,
        out_shape=jax.ShapeDtypeStruct((B // 8, 2, 8, W0), jnp.int32),
    )(x)


def _sc_embed():
    mesh = plsc.VectorSubcoreMesh(core_axis_name="c", subcore_axis_name="s")

    @functools.partial(
        pl.kernel,
        mesh=mesh,
        compiler_params=pltpu.CompilerParams(use_tc_tiling_on_sc=False),
        out_type=jax.ShapeDtypeStruct((NW, NCH, CHUNK, EMBED), jnp.float32),
        scratch_types=[
            pltpu.VMEM((BPW // 8, 2, 8, W0), jnp.int32),
            pltpu.VMEM((2, CHUNK, EMBED), jnp.float32),
            pltpu.VMEM((T, EMBED), jnp.float32),
            pltpu.SemaphoreType.DMA,
            pltpu.SemaphoreType.DMA,
            pltpu.SemaphoreType.DMA,
            pltpu.SemaphoreType.DMA,
        ],
    )
    def k(table_hbm, xp_hbm, pos_hbm, out_hbm,
          idx_v, rows_v, pos_v, gsem0, gsem1, osem0, osem1):
        wid = lax.axis_index("s") * 2 + lax.axis_index("c")
        gsem = (gsem0, gsem1)
        osem = (osem0, osem1)

        def fire_gathers(b, g):
            for q in range(BPC):
                bl = BPC * g + q
                bt = bl // 8
                b8 = bl % 8
                pltpu.async_copy(
                    table_hbm.at[idx_v.at[bt, 0, b8]],
                    rows_v.at[b, pl.ds(q * T, W0)],
                    gsem[b],
                )
                pltpu.async_copy(
                    table_hbm.at[idx_v.at[bt, 1, b8, pl.ds(0, W1)]],
                    rows_v.at[b, pl.ds(q * T + W0, W1)],
                    gsem[b],
                )

        def drain_gathers(b):
            pltpu.make_async_copy(
                out_hbm.at[wid, 0], rows_v.at[b], gsem[b]).wait()

        def fire_out(b, g):
            pltpu.async_copy(rows_v.at[b], out_hbm.at[wid, g], osem[b])

        def drain_out(b):
            pltpu.make_async_copy(
                rows_v.at[b], out_hbm.at[wid, 0], osem[b]).wait()

        def pos_add(b):
            def row_body(r, c):
                p0 = pos_v[r, pl.ds(0, 16)]
                p1 = pos_v[r, pl.ds(16, 16)]
                for q in range(BPC):
                    rr = q * T + r
                    rows_v[b, rr, pl.ds(0, 16)] = (
                        rows_v[b, rr, pl.ds(0, 16)] + p0)
                    rows_v[b, rr, pl.ds(16, 16)] = (
                        rows_v[b, rr, pl.ds(16, 16)] + p1)
                return c

            lax.fori_loop(0, T, row_body, 0)

        pltpu.sync_copy(pos_hbm, pos_v)
        pltpu.sync_copy(xp_hbm.at[pl.ds(wid * (BPW // 8), BPW // 8)], idx_v)
        fire_gathers(0, 0)

        def outer(i, carry):
            for b in range(2):
                g = 2 * i + b
                pb = 1 - b

                @pl.when(g + 1 < NCH)
                def _stage():
                    @pl.when(g >= 1)
                    def _free():
                        drain_out(pb)

                    fire_gathers(pb, g + 1)

                drain_gathers(b)
                pos_add(b)
                fire_out(b, g)
            return carry

        lax.fori_loop(0, NCH // 2, outer, 0)
        drain_out(0)
        drain_out(1)

    return k


def kernel(x, token_emb, pos_emb):
    xp = _repack_x(x.astype(jnp.int32))
    table = _detile_table(token_emb.T).reshape(PVOCAB, EMBED)
    out = _sc_embed()(table, xp, pos_emb)
    return out.reshape(B, T, EMBED)


# speedup vs baseline: 2.4343x; 1.4815x over previous
import functools

import jax
import jax.numpy as jnp
from jax import lax
from jax.experimental import pallas as pl
from jax.experimental.pallas import tpu as pltpu
from jax.experimental.pallas import tpu_sc as plsc

EMBED = 32
T = 200
B = 4096

NW = 32
BPW = B // NW
BPC = 4
NCH = BPW // BPC
CHUNK = BPC * T
W0, W1 = 128, T - 128


VOCAB = 1000000
DCOLS = 8192
NBLK = (VOCAB + DCOLS - 1) // DCOLS
PVOCAB = NBLK * DCOLS


def _detile_body(t_ref, y_ref):
    for k in range(DCOLS // 512):
        q = jnp.concatenate(
            [t_ref[:, pl.ds(512 * k + 128 * j, 128)] for j in range(4)],
            axis=0)
        y_ref[pl.ds(128 * k, 128), :] = q.T


def _detile_table(tokt):
    return pl.pallas_call(
        _detile_body,
        grid=(NBLK,),
        in_specs=[pl.BlockSpec((EMBED, DCOLS), lambda i: (0, i))],
        out_specs=pl.BlockSpec((DCOLS // 4, 128), lambda i: (i, 0)),
        out_shape=jax.ShapeDtypeStruct((PVOCAB * EMBED // 128, 128),
                                       jnp.float32),
    )(tokt)


def _perm(v):
    return ((v >> 9) << 9) | ((v & 127) << 2) | ((v >> 7) & 3)


def _repack_x_body(x_ref, y_ref):
    xp = _perm(x_ref[...])
    a = xp[:, :W0].reshape(B // 8, 8, W0)
    c = jnp.concatenate(
        [xp[:, W0:], jnp.zeros((B, W0 - W1), jnp.int32)], axis=1)
    y_ref[:, 0] = a
    y_ref[:, 1] = c.reshape(B // 8, 8, W0)


def _repack_x(x):
    return pl.pallas_call(
        _repack_x_body,
        out_shape=jax.ShapeDtypeStruct((B // 8, 2, 8, W0), jnp.int32),
    )(x)


def _sc_embed():
    mesh = plsc.VectorSubcoreMesh(core_axis_name="c", subcore_axis_name="s")

    @functools.partial(
        pl.kernel,
        mesh=mesh,
        compiler_params=pltpu.CompilerParams(use_tc_tiling_on_sc=False),
        out_type=jax.ShapeDtypeStruct((NW, NCH, CHUNK, EMBED), jnp.float32),
        scratch_types=[
            pltpu.VMEM((BPW // 8, 2, 8, W0), jnp.int32),
            pltpu.VMEM((2, CHUNK, EMBED), jnp.float32),
            pltpu.VMEM((T, EMBED), jnp.float32),
            pltpu.SemaphoreType.DMA,
            pltpu.SemaphoreType.DMA,
            pltpu.SemaphoreType.DMA,
            pltpu.SemaphoreType.DMA,
        ],
    )
    def k(table_hbm, xp_hbm, pos_hbm, out_hbm,
          idx_v, rows_v, pos_v, gsem0, gsem1, osem0, osem1):
        wid = lax.axis_index("s") * 2 + lax.axis_index("c")
        gsem = (gsem0, gsem1)
        osem = (osem0, osem1)

        def fire_gathers(b, g):
            for q in range(BPC):
                bl = BPC * g + q
                bt = bl // 8
                b8 = bl % 8
                pltpu.async_copy(
                    table_hbm.at[idx_v.at[bt, 0, b8]],
                    rows_v.at[b, pl.ds(q * T, W0)],
                    gsem[b],
                )
                pltpu.async_copy(
                    table_hbm.at[idx_v.at[bt, 1, b8, pl.ds(0, W1)]],
                    rows_v.at[b, pl.ds(q * T + W0, W1)],
                    gsem[b],
                )

        def drain_gathers(b):
            pltpu.make_async_copy(
                out_hbm.at[wid, 0], rows_v.at[b], gsem[b]).wait()

        def fire_out(b, g):
            pltpu.async_copy(rows_v.at[b], out_hbm.at[wid, g], osem[b])

        def drain_out(b):
            pltpu.make_async_copy(
                rows_v.at[b], out_hbm.at[wid, 0], osem[b]).wait()

        def pos_add(b):
            def row_body(r, c):
                p0 = pos_v[r, pl.ds(0, 16)]
                p1 = pos_v[r, pl.ds(16, 16)]
                for q in range(BPC):
                    rr = q * T + r
                    rows_v[b, rr, pl.ds(0, 16)] = (
                        rows_v[b, rr, pl.ds(0, 16)] + p0)
                    rows_v[b, rr, pl.ds(16, 16)] = (
                        rows_v[b, rr, pl.ds(16, 16)] + p1)
                return c

            lax.fori_loop(0, T, row_body, 0)

        pltpu.sync_copy(pos_hbm, pos_v)
        pltpu.sync_copy(xp_hbm.at[pl.ds(wid * (BPW // 8), BPW // 8)], idx_v)
        fire_gathers(0, 0)

        def outer(i, carry):
            for b in range(2):
                g = 2 * i + b
                pb = 1 - b

                @pl.when(g + 1 < NCH)
                def _stage():
                    @pl.when(g >= 1)
                    def _free():
                        drain_out(pb)

                    fire_gathers(pb, g + 1)

                drain_gathers(b)
                pos_add(b)
                fire_out(b, g)
            return carry

        lax.fori_loop(0, NCH // 2, outer, 0)
        drain_out(0)
        drain_out(1)

    return k


def kernel(x, token_emb, pos_emb):
    xp = _repack_x(x.astype(jnp.int32))
    table = _detile_table(token_emb.T).reshape(PVOCAB, EMBED)
    out = _sc_embed()(table, xp, pos_emb)
    return out.reshape(B, T, EMBED)
